# Initial kernel scaffold; baseline (speedup 1.0000x reference)
import jax
import jax.numpy as jnp
from jax.experimental import pallas as pl


def kernel(J_msg, b, msg_node, idx_msg_edge, mW1, mb1, mW2, mb2, mW3, mb3, aW1, ab1, aW2, ab2, Wih, bih, Whh, bhh, gW1, gb1, gW2, gb2, oW1, ob1, oW2, ob2, oW3, ob3):
    raise NotImplementedError("write your pallas kernel here")


# speedup vs baseline: 11.1622x; 11.1622x over previous
import functools

import jax
import jax.numpy as jnp
from jax import lax
from jax.experimental import pallas as pl
from jax.experimental.pallas import tpu as pltpu
from jax.experimental.pallas import tpu_sc as plsc

H = 128
L = 16
NC = 2
NS = 16
NW = NC * NS
DW = 144
SHIFT = 14
MASK14 = (1 << SHIFT) - 1

C_E = 16
EC_E = 10000
ECP_E = 10016
ZST_E = 626
CST_E = 625

C_N = 2
EC_N = 5000
ECP_N = 5008
ZST_N = 313
CST_N = 313

_mesh = lambda: plsc.VectorSubcoreMesh(core_axis_name="c", subcore_axis_name="s")
_SC_PARAMS = lambda: pltpu.CompilerParams(needs_layout_passes=False,
                                          use_tc_tiling_on_sc=False)


def _wid():
  return lax.axis_index("s") * NC + lax.axis_index("c")


def _iota():
  return lax.iota(jnp.int32, L)


def _sc_prep(mn_flat, b1d):
  E = mn_flat.shape[0] // 2
  N = b1d.shape[0]
  NPT = E // NW

  @functools.partial(
      pl.kernel, mesh=_mesh(), compiler_params=_SC_PARAMS(),
      out_type=(jax.ShapeDtypeStruct((E,), jnp.float32),
                jax.ShapeDtypeStruct((E,), jnp.float32)),
      scratch_types=[pltpu.VMEM((2 * NPT,), jnp.int32),
                     pltpu.VMEM((N,), jnp.float32),
                     pltpu.VMEM((NPT,), jnp.float32),
                     pltpu.VMEM((NPT,), jnp.float32)])
  def kfn(mn_hbm, b_hbm, bin_hbm, bout_hbm, mn_v, b_v, bi_v, bo_v):
    t = _wid()
    base = t * NPT
    pltpu.sync_copy(mn_hbm.at[pl.ds(2 * base, 2 * NPT)], mn_v)
    pltpu.sync_copy(b_hbm, b_v)

    def body(g, carry):
      rvec = 2 * (_iota() + g * L)
      n0 = plsc.load_gather(mn_v, [rvec])
      n1 = plsc.load_gather(mn_v, [rvec + 1])
      bi_v[pl.ds(g * L, L)] = plsc.load_gather(b_v, [n0])
      bo_v[pl.ds(g * L, L)] = plsc.load_gather(b_v, [n1])
      return carry

    lax.fori_loop(0, NPT // L, body, 0)
    pltpu.sync_copy(bi_v, bin_hbm.at[pl.ds(base, NPT)])
    pltpu.sync_copy(bo_v, bout_hbm.at[pl.ds(base, NPT)])

  return kfn(mn_flat, b1d)


def _sc_partition(pairs_flat, self_src, n_chunks, ec, cap):
  NP = pairs_flat.shape[0] // 2
  NPT = NP // NW
  GB = NPT // L
  dump_packed = ec

  @functools.partial(
      pl.kernel, mesh=_mesh(), compiler_params=_SC_PARAMS(),
      out_type=(jax.ShapeDtypeStruct((n_chunks * NW * cap,), jnp.int32),
                jax.ShapeDtypeStruct((NW, L), jnp.int32)),
      scratch_types=[pltpu.VMEM((2 * NPT,), jnp.int32),
                     pltpu.VMEM((cap,), jnp.int32),
                     pltpu.VMEM((L,), jnp.int32)])
  def kfn(pr_hbm, pairs_hbm, counts_hbm, p_v, sbuf, cbuf):
    t = _wid()
    base = t * NPT
    pltpu.sync_copy(pr_hbm.at[pl.ds(2 * base, 2 * NPT)], p_v)
    cvec = jnp.zeros((L,), jnp.int32)
    for c in range(n_chunks):
      lo = c * ec

      def body(g, cursor):
        rvec = _iota() + g * L
        dst_g = plsc.load_gather(p_v, [2 * rvec + 1])
        if self_src:
          src_g = base + rvec
        else:
          src_g = plsc.load_gather(p_v, [2 * rvec])
        m = (dst_g >= lo) & (dst_g < lo + ec)
        packed = lax.shift_left(src_g, SHIFT) | (dst_g - lo)
        plsc.store_compressed(sbuf.at[pl.ds(cursor, L)], packed, mask=m)
        return cursor + jnp.sum(m.astype(jnp.int32))

      cursor = lax.fori_loop(0, GB, body, jnp.int32(0))
      dumpv = jnp.full((L,), dump_packed, jnp.int32)
      sbuf[pl.ds(cursor, L)] = dumpv
      sbuf[pl.ds(cursor + L, L)] = dumpv
      cnt32 = (cursor + 31) & (-32)
      cvec = jnp.where(_iota() == c, cnt32, cvec)
      pltpu.sync_copy(sbuf, pairs_hbm.at[pl.ds((c * NW + t) * cap, cap)])
    cbuf[...] = cvec
    pltpu.sync_copy(cbuf, counts_hbm.at[t])

  return kfn(pairs_flat)


def _sc_agg(w_rows, pairs, counts, zrows, n_chunks, cpc, ec, ecp, zst, cst,
            cap, out_rows, full_copy):
  @functools.partial(
      pl.kernel, mesh=_mesh(), compiler_params=_SC_PARAMS(),
      out_type=jax.ShapeDtypeStruct((out_rows, DW), jnp.float32),
      scratch_types=[pltpu.VMEM_SHARED((ecp, DW), jnp.float32),
                     pltpu.VMEM((32,), jnp.int32),
                     pltpu.VMEM((32,), jnp.int32),
                     pltpu.VMEM((32, DW), jnp.float32),
                     pltpu.VMEM((NW, L), jnp.int32),
                     pltpu.SemaphoreType.DMA])
  def kfn(w_hbm, pairs_hbm, counts_hbm, z_hbm, out_hbm,
          acc, pbuf, sbufi, rows, cnt_v, sem):
    q = lax.axis_index("c")
    s = lax.axis_index("s")
    pltpu.sync_copy(counts_hbm, cnt_v)
    for ci in range(cpc):
      c = q * cpc + ci
      pltpu.sync_copy(z_hbm.at[pl.ds(0, zst)], acc.at[pl.ds(s * zst, zst)])
      plsc.subcore_barrier()
      for j in range(2):
        t = s * 2 + j
        row = cnt_v[t]
        nb = jnp.max(jnp.where(_iota() == c, row, 0)) // 32
        pbase = (c * NW + t) * cap

        def bbody(i, carry):
          pltpu.sync_copy(pairs_hbm.at[pl.ds(pbase + i * 32, 32)], pbuf)
          for u in range(2):
            pv = pbuf[pl.ds(u * L, L)]
            sbufi[pl.ds(u * L, L)] = lax.shift_right_logical(pv, SHIFT)
          pltpu.async_copy(w_hbm.at[sbufi], rows, sem).wait()
          for u in range(2):
            dstv = pbuf[pl.ds(u * L, L)] & MASK14
            pltpu.sync_copy(rows.at[pl.ds(u * L, L), :], acc.at[dstv],
                            add=True)
          return carry

        lax.fori_loop(0, nb, bbody, 0)
      plsc.subcore_barrier()
      out_base = c * ecp if full_copy else c * ec
      pltpu.sync_copy(acc.at[pl.ds(s * cst, cst), :],
                      out_hbm.at[pl.ds(out_base + s * cst, cst), :])
      plsc.subcore_barrier()

  return kfn(w_rows, pairs, counts, zrows)


def _dot(a, b):
  return lax.dot(a, b, preferred_element_type=jnp.float32)


def _ff_contrib(w8, bj):
  b_in, b_out, Jv = bj[:, 0:1], bj[:, 1:2], bj[:, 2:3]
  return (b_in * (w8[0:1] - w8[1:2]) + b_out * (w8[2:3] - w8[3:4])
          + Jv * (w8[4:5] - w8[5:6] - w8[6:7] + w8[7:8]))


def _softmax(x):
  m = jnp.max(x, axis=1, keepdims=True)
  e = jnp.exp(x - m)
  return e / jnp.sum(e, axis=1, keepdims=True)


def _msg_gru(state_agg, st, bj, mW1, mb1, mW2, mb2, mW3, mb3,
             Wih, bih, Whh, bhh):
  h1 = jnp.maximum(_dot(state_agg, mW1[:H]) + _ff_contrib(mW1[H:], bj) + mb1,
                   0.0)
  h2 = jnp.maximum(_dot(h1, mW2) + mb2, 0.0)
  msg = _softmax(_dot(h2, mW3) + mb3)
  gi = _dot(msg, Wih) + bih
  gh = (_dot(st, Whh) + bhh) if st is not None else bhh
  r = jax.nn.sigmoid(gi[:, :H] + gh[:, :H])
  z = jax.nn.sigmoid(gi[:, H:2 * H] + gh[:, H:2 * H])
  n = jnp.tanh(gi[:, 2 * H:] + r * gh[:, 2 * H:])
  upd = (1.0 - z) * n
  if st is not None:
    upd = upd + z * st
  return _softmax(upd)


def _full(shape):
  return pl.BlockSpec(shape, lambda i: (0,) * len(shape))


def _rows(be, w):
  return pl.BlockSpec((be, w), lambda i: (i, 0))


def _k_first(bioJ, mW1, mb1, mW2, mb2, mW3, mb3, Wih, bih, Whh, bhh, E, BE):
  def body(bj_ref, mW1_r, mb1_r, mW2_r, mb2_r, mW3_r, mb3_r,
           Wih_r, bih_r, Whh_r, bhh_r, out_ref):
    zero_agg = jnp.zeros((BE, H), jnp.float32)
    out_ref[...] = _msg_gru(zero_agg, None, bj_ref[...],
                            mW1_r[...], mb1_r[...], mW2_r[...], mb2_r[...],
                            mW3_r[...], mb3_r[...], Wih_r[...], bih_r[...],
                            Whh_r[...], bhh_r[...])

  return pl.pallas_call(
      body, grid=(E // BE,),
      in_specs=[_rows(BE, 4), _full((H + 8, 64)), _full((1, 64)),
                _full((64, 64)), _full((1, 64)), _full((64, H)),
                _full((1, H)), _full((H, 3 * H)), _full((1, 3 * H)),
                _full((H, 3 * H)), _full((1, 3 * H))],
      out_specs=_rows(BE, H),
      out_shape=jax.ShapeDtypeStruct((E, H), jnp.float32),
  )(bioJ, mW1, mb1, mW2, mb2, mW3, mb3, Wih, bih, Whh, bhh)


def _k_att(state, bioJ, aW1, ab1, aW2, ab2, E, BE):
  def body(st_ref, bj_ref, aW1_r, ab1_r, aW2_r, ab2_r, out_ref):
    st = st_ref[...]
    aw = aW1_r[...]
    h = jnp.maximum(_dot(st, aw[:H]) + _ff_contrib(aw[H:], bj_ref[...])
                    + ab1_r[...], 0.0)
    att = jnp.exp(jax.nn.sigmoid(_dot(h, aW2_r[...]) + ab2_r[...]))
    out_ref[...] = jnp.concatenate(
        [att * st, att, jnp.zeros((BE, DW - H - 1), jnp.float32)], axis=1)

  return pl.pallas_call(
      body, grid=(E // BE,),
      in_specs=[_rows(BE, H), _rows(BE, 4), _full((H + 8, 64)),
                _full((1, 64)), _full((64, 1)), _full((1, 1))],
      out_specs=_rows(BE, DW),
      out_shape=jax.ShapeDtypeStruct((E, DW), jnp.float32),
  )(state, bioJ, aW1, ab1, aW2, ab2)


def _k_upd(acc, state, bioJ, mW1, mb1, mW2, mb2, mW3, mb3,
           Wih, bih, Whh, bhh, E, BE):
  def body(acc_ref, st_ref, bj_ref, mW1_r, mb1_r, mW2_r, mb2_r, mW3_r, mb3_r,
           Wih_r, bih_r, Whh_r, bhh_r, out_ref):
    acc = acc_ref[...]
    norm = acc[:, H:H + 1]
    state_agg = jnp.where(norm > 0.5, acc[:, :H] / norm, 0.0)
    out_ref[...] = _msg_gru(state_agg, st_ref[...], bj_ref[...],
                            mW1_r[...], mb1_r[...], mW2_r[...], mb2_r[...],
                            mW3_r[...], mb3_r[...], Wih_r[...], bih_r[...],
                            Whh_r[...], bhh_r[...])

  return pl.pallas_call(
      body, grid=(E // BE,),
      in_specs=[_rows(BE, DW), _rows(BE, H), _rows(BE, 4),
                _full((H + 8, 64)), _full((1, 64)), _full((64, 64)),
                _full((1, 64)), _full((64, H)), _full((1, H)),
                _full((H, 3 * H)), _full((1, 3 * H)), _full((H, 3 * H)),
                _full((1, 3 * H))],
      out_specs=_rows(BE, H),
      out_shape=jax.ShapeDtypeStruct((E, H), jnp.float32),
  )(acc, state, bioJ, mW1, mb1, mW2, mb2, mW3, mb3, Wih, bih, Whh, bhh)


def _k_gout(state, gW1, gb1, gW2, gb2, E, BE):
  def body(st_ref, gW1_r, gb1_r, gW2_r, gb2_r, out_ref):
    st = st_ref[...]
    h = jnp.maximum(_dot(st, gW1_r[...]) + gb1_r[...], 0.0)
    att = jnp.exp(jax.nn.sigmoid(_dot(h, gW2_r[...]) + gb2_r[...]))
    out_ref[...] = jnp.concatenate(
        [att * st, att, jnp.zeros((BE, DW - H - 1), jnp.float32)], axis=1)

  return pl.pallas_call(
      body, grid=(E // BE,),
      in_specs=[_rows(BE, H), _full((H, 64)), _full((1, 64)),
                _full((64, 1)), _full((1, 1))],
      out_specs=_rows(BE, DW),
      out_shape=jax.ShapeDtypeStruct((E, DW), jnp.float32),
  )(state, gW1, gb1, gW2, gb2)


def _k_out(accN, b_pad, oW1, ob1, oW2, ob2, oW3, ob3):
  NP2 = 2 * ECP_N
  BN = ECP_N

  def body(acc_ref, b_ref, oW1_r, ob1_r, oW2_r, ob2_r, oW3_r, ob3_r, out_ref):
    acc = acc_ref[...]
    norm = acc[:, H:H + 1]
    out_agg = jnp.where(norm > 0.5, acc[:, :H] / norm, 0.0)
    ow1 = oW1_r[...]
    bv = b_ref[...]
    h = jnp.maximum(_dot(out_agg, ow1[:H]) + bv * (ow1[H:H + 1] - ow1[H + 1:])
                    + ob1_r[...], 0.0)
    h2 = jnp.maximum(_dot(h, oW2_r[...]) + ob2_r[...], 0.0)
    y = _dot(h2, oW3_r[...]) + ob3_r[...]
    m = jnp.max(y, axis=1, keepdims=True)
    ys = y - m
    out_ref[...] = ys - jnp.log(jnp.sum(jnp.exp(ys), axis=1, keepdims=True))

  return pl.pallas_call(
      body, grid=(NP2 // BN,),
      in_specs=[_rows(BN, DW), _rows(BN, 1), _full((H + 2, 64)),
                _full((1, 64)), _full((64, 64)), _full((1, 64)),
                _full((64, 2)), _full((1, 2))],
      out_specs=_rows(BN, 2),
      out_shape=jax.ShapeDtypeStruct((NP2, 2), jnp.float32),
  )(accN, b_pad, oW1, ob1, oW2, ob2, oW3, ob3)


NUM_PROP = 5


def kernel(J_msg, b, msg_node, idx_msg_edge, mW1, mb1, mW2, mb2, mW3, mb3,
           aW1, ab1, aW2, ab2, Wih, bih, Whh, bhh, gW1, gb1, gW2, gb2,
           oW1, ob1, oW2, ob2, oW3, ob3):
  E = msg_node.shape[0]
  N = b.shape[0]
  M = idx_msg_edge.shape[0]
  BE = 2000

  cap_e = (M // NW) + 32
  cap_n = ((E // NW) + 63) & -32

  b1d = b.reshape(N)
  r1 = lambda x: x.reshape(1, -1)
  mb1r, mb2r, mb3r = r1(mb1), r1(mb2), r1(mb3)
  ab1r, ab2r = r1(ab1), r1(ab2)
  bihr, bhhr = r1(bih), r1(bhh)
  gb1r, gb2r = r1(gb1), r1(gb2)
  ob1r, ob2r, ob3r = r1(ob1), r1(ob2), r1(ob3)

  b_in, b_out = _sc_prep(msg_node.reshape(-1), b1d)
  bioJ = jnp.concatenate(
      [b_in.reshape(E, 1), b_out.reshape(E, 1), J_msg,
       jnp.zeros((E, 1), jnp.float32)], axis=1)

  pairsE, countsE = _sc_partition(idx_msg_edge.reshape(-1), False,
                                  C_E, EC_E, cap_e)
  pairsN, countsN = _sc_partition(msg_node.reshape(-1), True,
                                  C_N, EC_N, cap_n)
  zrows = jnp.zeros((ZST_E, DW), jnp.float32)

  state = _k_first(bioJ, mW1, mb1r, mW2, mb2r, mW3, mb3r,
                   Wih, bihr, Whh, bhhr, E, BE)
  for _ in range(NUM_PROP - 1):
    w144 = _k_att(state, bioJ, aW1, ab1r, aW2, ab2r, E, BE)
    acc = _sc_agg(w144, pairsE, countsE, zrows, C_E, C_E // NC, EC_E, ECP_E,
                  ZST_E, CST_E, cap_e, E, False)
    state = _k_upd(acc, state, bioJ, mW1, mb1r, mW2, mb2r, mW3, mb3r,
                   Wih, bihr, Whh, bhhr, E, BE)

  w144f = _k_gout(state, gW1, gb1r, gW2, gb2r, E, BE)
  accN = _sc_agg(w144f, pairsN, countsN, zrows, C_N, C_N // NC, EC_N, ECP_N,
                 ZST_N, CST_N, cap_n, 2 * ECP_N, True)

  b_pad = jnp.concatenate(
      [b[:EC_N], jnp.zeros((ECP_N - EC_N, 1), jnp.float32),
       b[EC_N:], jnp.zeros((ECP_N - EC_N, 1), jnp.float32)], axis=0)
  ypad = _k_out(accN, b_pad, oW1, ob1r, oW2, ob2r, oW3, ob3r)
  return jnp.concatenate([ypad[:EC_N], ypad[ECP_N:ECP_N + EC_N]], axis=0)
